# baseline (speedup 1.0000x reference)
import jax
import jax.numpy as jnp
from jax.experimental import pallas as pl

_NUM_ANCHORS = 3
_NUM_CLASSES = 80
_STRIDE = 16.0
_G = 64
_S = _G * _G
_CH = _NUM_CLASSES + 5
_ANCHORS = ((0.12, 0.16), (0.30, 0.38), (0.72, 0.55))
_AW16 = tuple(float(jnp.float32(a[0]) * _G * _STRIDE) for a in _ANCHORS)
_AH16 = tuple(float(jnp.float32(a[1]) * _G * _STRIDE) for a in _ANCHORS)


def _yolo_body(x_ref, boxes_ref, conf_ref, cls_ref):
    a = pl.program_id(0) % _NUM_ANCHORS
    xf = x_ref[0]

    head = xf[0:8]
    r = jax.lax.broadcasted_iota(jnp.int32, (8, _S), 0)
    head_act = jnp.where((r == 2) | (r == 3), jnp.exp(head),
                         jax.nn.sigmoid(head))
    tail_act = jax.nn.sigmoid(xf[8:_CH])
    act = jnp.concatenate([head_act, tail_act], axis=0)

    conf_ref[0] = act[4:5]

    act_t = act.T

    s = jax.lax.broadcasted_iota(jnp.int32, (_S, 1), 0)
    gx = (s & (_G - 1)).astype(jnp.float32)
    gy = (s >> 6).astype(jnp.float32)
    aw16 = jnp.where(a == 0, _AW16[0], jnp.where(a == 1, _AW16[1], _AW16[2]))
    ah16 = jnp.where(a == 0, _AH16[0], jnp.where(a == 1, _AH16[1], _AH16[2]))
    col = jax.lax.broadcasted_iota(jnp.int32, (_S, 4), 1)
    offs = jnp.where(col == 0, gx, jnp.where(col == 1, gy, 0.0))
    scale = jnp.where(col == 0, _STRIDE,
                      jnp.where(col == 1, _STRIDE,
                                jnp.where(col == 2, aw16, ah16)))
    boxes_ref[0] = (act_t[:, 0:4] + offs) * scale

    cls_ref[0] = act_t[:, 5:_CH]


def kernel(x):
    B = x.shape[0]
    n = B * _NUM_ANCHORS
    xr = x.reshape(n, _CH, _S)

    boxes, conf, cls = pl.pallas_call(
        _yolo_body,
        grid=(n,),
        in_specs=[pl.BlockSpec((1, _CH, _S), lambda i: (i, 0, 0))],
        out_specs=[
            pl.BlockSpec((1, _S, 4), lambda i: (i, 0, 0)),
            pl.BlockSpec((1, 1, _S), lambda i: (i, 0, 0)),
            pl.BlockSpec((1, _S, _NUM_CLASSES), lambda i: (i, 0, 0)),
        ],
        out_shape=[
            jax.ShapeDtypeStruct((n, _S, 4), jnp.float32),
            jax.ShapeDtypeStruct((n, 1, _S), jnp.float32),
            jax.ShapeDtypeStruct((n, _S, _NUM_CLASSES), jnp.float32),
        ],
    )(xr)

    out_boxes = boxes.reshape(B, _NUM_ANCHORS * _S, 4)
    out_conf = conf.reshape(B, _NUM_ANCHORS * _S, 1)
    out_cls = cls.reshape(B, _NUM_ANCHORS * _S, _NUM_CLASSES)
    return (out_boxes, out_conf, out_cls)


# speedup vs baseline: 1.0032x; 1.0032x over previous
import jax
import jax.numpy as jnp
import numpy as np
from jax.experimental import pallas as pl

_NUM_ANCHORS = 3
_NUM_CLASSES = 80
_STRIDE = 16.0
_G = 64
_S = _G * _G
_CH = _NUM_CLASSES + 5
_ANCHORS = ((0.12, 0.16), (0.30, 0.38), (0.72, 0.55))
_AW16 = tuple(float(np.float32(a[0]) * _G * _STRIDE) for a in _ANCHORS)
_AH16 = tuple(float(np.float32(a[1]) * _G * _STRIDE) for a in _ANCHORS)


def _yolo_body(x_ref, boxes_ref, conf_ref, cls_ref):
    a = pl.program_id(0) % _NUM_ANCHORS
    xf = x_ref[0]

    head = xf[0:8]
    r = jax.lax.broadcasted_iota(jnp.int32, (8, _S), 0)
    head_act = jnp.where((r == 2) | (r == 3), jnp.exp(head),
                         jax.nn.sigmoid(head))
    tail_act = jax.nn.sigmoid(xf[8:_CH])
    act = jnp.concatenate([head_act, tail_act], axis=0)

    conf_ref[0] = act[4:5]

    act_t = act.T

    s = jax.lax.broadcasted_iota(jnp.int32, (_S, 1), 0)
    gx = (s & (_G - 1)).astype(jnp.float32)
    gy = (s >> 6).astype(jnp.float32)
    aw16 = jnp.where(a == 0, _AW16[0], jnp.where(a == 1, _AW16[1], _AW16[2]))
    ah16 = jnp.where(a == 0, _AH16[0], jnp.where(a == 1, _AH16[1], _AH16[2]))
    col = jax.lax.broadcasted_iota(jnp.int32, (_S, 4), 1)
    offs = jnp.where(col == 0, gx, jnp.where(col == 1, gy, 0.0))
    scale = jnp.where(col == 0, _STRIDE,
                      jnp.where(col == 1, _STRIDE,
                                jnp.where(col == 2, aw16, ah16)))
    boxes_ref[0] = (act_t[:, 0:4] + offs) * scale

    cls_ref[0] = act_t[:, 5:_CH]


def kernel(x):
    B = x.shape[0]
    n = B * _NUM_ANCHORS
    xr = x.reshape(n, _CH, _S)

    boxes, conf, cls = pl.pallas_call(
        _yolo_body,
        grid=(n,),
        in_specs=[pl.BlockSpec((1, _CH, _S), lambda i: (i, 0, 0))],
        out_specs=[
            pl.BlockSpec((1, _S, 4), lambda i: (i, 0, 0)),
            pl.BlockSpec((1, 1, _S), lambda i: (i, 0, 0)),
            pl.BlockSpec((1, _S, _NUM_CLASSES), lambda i: (i, 0, 0)),
        ],
        out_shape=[
            jax.ShapeDtypeStruct((n, _S, 4), jnp.float32),
            jax.ShapeDtypeStruct((n, 1, _S), jnp.float32),
            jax.ShapeDtypeStruct((n, _S, _NUM_CLASSES), jnp.float32),
        ],
    )(xr)

    out_boxes = boxes.reshape(B, _NUM_ANCHORS * _S, 4)
    out_conf = conf.reshape(B, _NUM_ANCHORS * _S, 1)
    out_cls = cls.reshape(B, _NUM_ANCHORS * _S, _NUM_CLASSES)
    return (out_boxes, out_conf, out_cls)
